# Initial kernel scaffold; baseline (speedup 1.0000x reference)
import jax
import jax.numpy as jnp
from jax.experimental import pallas as pl


def kernel(inputs, theta):
    raise NotImplementedError("write your pallas kernel here")


# speedup vs baseline: 1.1809x; 1.1809x over previous
import functools

import jax
import jax.numpy as jnp
from jax import lax
from jax.experimental import pallas as pl
from jax.experimental.pallas import tpu as pltpu
from jax.experimental.pallas import tpu_sc as plsc

B = 16384
D = 32
NCOLS = 12
EPS = 1e-05

NC = 2
NS = 16
NW = NC * NS
RPW = B // NW
BLK = 64
NBLK = RPW // BLK
CHUNK = 96
NCHUNK = BLK * NCOLS // CHUNK
NBUF = 2
NGRP = BLK // 16
NSTAT = 24


def _sc_body(idx_hbm, theta_hbm, out_hbm, idx_v, gbuf0, gbuf1, res, sem0, sem1):
  sems = (sem0, sem1)
  gbufs = (gbuf0, gbuf1)
  wid = lax.axis_index("s") * NC + lax.axis_index("c")
  base = wid * RPW
  iota = lax.iota(jnp.int32, 16)

  pltpu.sync_copy(idx_hbm.at[pl.ds(wid * (NBLK * NCHUNK), NBLK * NCHUNK)], idx_v)

  def fire(c):
    buf = c % NBUF
    handles = []
    for k in range(NCHUNK):
      handles.append(
          pltpu.async_copy(
              theta_hbm.at[idx_v.at[c * NCHUNK + k]],
              gbufs[buf].at[pl.ds(k * CHUNK, CHUNK)],
              sems[buf],
          )
      )
    return handles

  def compute(c):
    gb = gbufs[c % NBUF]
    for g in range(NGRP):
      grow = (g * 16 + iota) * NCOLS
      zero = jnp.zeros((16,), jnp.float32)

      def make_pass(roles, with_su):
        nr = len(roles)

        def body(d, acc):
          dcol = jnp.zeros((16,), jnp.int32) + d
          u = plsc.load_gather(gb, [grow, dcol])
          out = list(acc)
          k0 = 0
          if with_su:
            out[0] = acc[0] + u * u
            k0 = 1
          for i, r in enumerate(roles):
            x = plsc.load_gather(gb, [grow + r, dcol])
            out[k0 + i] = acc[k0 + i] + x * x
            t = u - x
            out[k0 + nr + i] = acc[k0 + nr + i] + t * t
          return out

        n = nr * 2 + (1 if with_su else 0)
        return lax.fori_loop(0, D, body, [zero] * n)

      ra = make_pass([1, 2, 3, 4, 5, 6], True)
      rb = make_pass([7, 8, 9, 10, 11], False)

      rows = c * BLK + g * 16 + iota

      def store(val, col):
        plsc.store_scatter(res, [rows, jnp.full((16,), col, jnp.int32)], val)

      store(ra[0], 0)
      for i, r in enumerate([1, 2, 3, 4, 5, 6]):
        store(ra[1 + i], r)
        store(ra[7 + i], 11 + r)
      for i, r in enumerate([7, 8, 9, 10, 11]):
        store(rb[i], r)
        store(rb[5 + i], 11 + r)

  pending = fire(0)
  for c in range(NBLK):
    nxt = fire(c + 1) if c + 1 < NBLK else []
    for h in pending:
      h.wait()
    compute(c)
    pending = nxt

  pltpu.sync_copy(res, out_hbm.at[pl.ds(base, RPW)])


@jax.jit
def _sc_stats(idx2d, theta):
  mesh = plsc.VectorSubcoreMesh(core_axis_name="c", subcore_axis_name="s")
  return pl.kernel(
      _sc_body,
      out_type=jax.ShapeDtypeStruct((B, NSTAT), jnp.float32),
      mesh=mesh,
      compiler_params=pltpu.CompilerParams(
          needs_layout_passes=False, use_tc_tiling_on_sc=False
      ),
      scratch_types=[
          pltpu.VMEM((NBLK * NCHUNK, CHUNK), jnp.int32),
          pltpu.VMEM((BLK * NCOLS, D), jnp.float32),
          pltpu.VMEM((BLK * NCOLS, D), jnp.float32),
          pltpu.VMEM((RPW, NSTAT), jnp.float32),
          pltpu.SemaphoreType.DMA,
          pltpu.SemaphoreType.DMA,
      ],
  )(idx2d, theta)


def _finish_body(s_ref, uv_ref, uvp_ref):
  s = s_ref[...]
  su = s[:, 0:1]
  sx = s[:, 1:12]
  dx = s[:, 12:23]
  omu = 1.0 - jnp.clip(su, 0.0, 1.0 - EPS)
  omx = 1.0 - jnp.clip(sx, 0.0, 1.0 - EPS)
  num = jnp.sqrt(dx + EPS)
  t = 1.0 + 2.0 * num / (omu * omx)
  dist = jnp.log(t + jnp.sqrt((t - 1.0) * (t + 1.0)))
  uv_ref[...] = dist[:, 0:1]
  uvp_ref[...] = dist[:, 1:11]


@jax.jit
def _tc_finish(stats):
  blk = 2048
  grid = B // blk
  return pl.pallas_call(
      _finish_body,
      grid=(grid,),
      in_specs=[pl.BlockSpec((blk, NSTAT), lambda i: (i, 0))],
      out_specs=[
          pl.BlockSpec((blk, 1), lambda i: (i, 0)),
          pl.BlockSpec((blk, 10), lambda i: (i, 0)),
      ],
      out_shape=[
          jax.ShapeDtypeStruct((B, 1), jnp.float32),
          jax.ShapeDtypeStruct((B, 10), jnp.float32),
      ],
  )(stats)


def kernel(inputs, theta):
  idx2d = inputs.reshape(B * NCOLS // CHUNK, CHUNK)
  stats = _sc_stats(idx2d, theta)
  uv, uvp = _tc_finish(stats)
  return uv, uvp
